# baseline (speedup 1.0000x reference)
import functools

import jax
import jax.numpy as jnp
from jax import lax
from jax.experimental import pallas as pl
from jax.experimental.pallas import tpu as pltpu
from jax.experimental.pallas import tpu_sc as plsc

_B = 16384
_SED = 32
_PED = 64
_NMODE_PAD = 16

_NW = 32
_ROWS_W = _B // _NW
_CHUNK = 128
_NCHUNK = _ROWS_W // _CHUNK

_BLK = 2048


def _sc_gather(user_table, idx2d):
    mesh = plsc.VectorSubcoreMesh(core_axis_name="c", subcore_axis_name="s")

    @functools.partial(
        pl.kernel,
        mesh=mesh,
        compiler_params=pltpu.CompilerParams(use_tc_tiling_on_sc=False),
        out_type=jax.ShapeDtypeStruct((_B, _SED), jnp.float32),
        scratch_types=[
            pltpu.VMEM((_NCHUNK, _CHUNK), jnp.int32),
            pltpu.VMEM((_NCHUNK, _CHUNK, _SED), jnp.float32),
            pltpu.SemaphoreType.DMA,
        ],
    )
    def gather_kernel(table_hbm, idx_hbm, out_hbm, idx_v, rows_v, sem):
        wid = lax.axis_index("s") * 2 + lax.axis_index("c")
        r0 = wid * _NCHUNK
        pltpu.sync_copy(idx_hbm.at[pl.ds(r0, _NCHUNK)], idx_v)
        copies = [
            pltpu.async_copy(table_hbm.at[idx_v.at[j]], rows_v.at[j], sem)
            for j in range(_NCHUNK)
        ]
        for c in copies:
            c.wait()
        for j in range(_NCHUNK):
            pltpu.sync_copy(
                rows_v.at[j], out_hbm.at[pl.ds((r0 + j) * _CHUNK, _CHUNK)]
            )

    return gather_kernel(user_table, idx2d)


def _tc_body(ug_ref, mode_ref, ts_ref, mt_ref, wu_ref, wm_ref, wti_ref,
             wt_ref, bt_ref, bp_ref, out_ref):
    mo = jnp.dot(mt_ref[...], wm_ref[...], preferred_element_type=jnp.float32)
    wc = jnp.dot(wti_ref[...], wt_ref[...], preferred_element_type=jnp.float32)
    bias = (
        jnp.dot(bt_ref[...], wt_ref[...], preferred_element_type=jnp.float32)
        + bp_ref[...]
    )
    iota = lax.broadcasted_iota(jnp.int32, (_BLK, _NMODE_PAD), 1)
    oh = (mode_ref[...] == iota).astype(jnp.float32)
    acc = jnp.dot(ug_ref[...], wu_ref[...], preferred_element_type=jnp.float32)
    acc = acc + jnp.dot(oh, mo, preferred_element_type=jnp.float32)
    acc = acc + jnp.dot(ts_ref[...], wc, preferred_element_type=jnp.float32)
    out_ref[...] = acc + bias


def _tc_dense(ug, mode2d, ts, mt16, Wu, Wm, W_time, Wt, bt2d, bp2d):
    return pl.pallas_call(
        _tc_body,
        grid=(_B // _BLK,),
        in_specs=[
            pl.BlockSpec((_BLK, _SED), lambda i: (i, 0)),
            pl.BlockSpec((_BLK, 1), lambda i: (i, 0)),
            pl.BlockSpec((_BLK, 6), lambda i: (i, 0)),
            pl.BlockSpec((_NMODE_PAD, _SED), lambda i: (0, 0)),
            pl.BlockSpec((_SED, _PED), lambda i: (0, 0)),
            pl.BlockSpec((_SED, _PED), lambda i: (0, 0)),
            pl.BlockSpec((6, _SED), lambda i: (0, 0)),
            pl.BlockSpec((_SED, _PED), lambda i: (0, 0)),
            pl.BlockSpec((1, _SED), lambda i: (0, 0)),
            pl.BlockSpec((1, _PED), lambda i: (0, 0)),
        ],
        out_specs=pl.BlockSpec((_BLK, _PED), lambda i: (i, 0)),
        out_shape=jax.ShapeDtypeStruct((_B, _PED), jnp.float32),
    )(ug, mode2d, ts, mt16, Wu, Wm, W_time, Wt, bt2d, bp2d)


def kernel(user_id, transport_mode, timestamp, user_table, mode_table,
           W_time, b_time, W_pref, b_pref):
    idx2d = user_id.astype(jnp.int32).reshape(_B // _CHUNK, _CHUNK)
    ug = _sc_gather(user_table, idx2d)
    mt16 = jnp.zeros((_NMODE_PAD, _SED), jnp.float32).at[:12].set(mode_table)
    Wu = W_pref[:_SED]
    Wm = W_pref[_SED:2 * _SED]
    Wt = W_pref[2 * _SED:]
    mode2d = transport_mode.astype(jnp.int32).reshape(_B, 1)
    return _tc_dense(ug, mode2d, timestamp, mt16, Wu, Wm, W_time, Wt,
                     b_time.reshape(1, _SED), b_pref.reshape(1, _PED))
